# baseline (speedup 1.0000x reference)
import jax
import jax.numpy as jnp
from jax.experimental import pallas as pl
from jax.experimental.pallas import tpu as pltpu

_N = 10000
_D = 128
_BM = 200
_NB = _N // _BM
_BF = jnp.bfloat16


def _dot(a, b):
    return jax.lax.dot_general(a, b, (((1,), (0,)), ((), ())),
                               preferred_element_type=jnp.float32)


def _body(x_ref, g_ref, w1_ref, sw1_ref, b1_ref, w2_ref, sw2_ref, b2_ref,
          o_ref, s1_scr, p1_scr, s2_scr, p2_scr):
    i = pl.program_id(0)

    @pl.when(i == 0)
    def _stage_a():
        x = x_ref[...].astype(_BF)
        s1_scr[...] = (_dot(x, w1_ref[...].astype(_BF))
                       + b1_ref[...]).astype(_BF)
        p1_scr[...] = _dot(x, sw1_ref[...].astype(_BF)).astype(_BF)

    @pl.when((i >= 1) & (i <= _NB))
    def _pass_1():
        r = (i - 1) * _BM
        g = g_ref[...].astype(_BF)
        h = jnp.maximum(
            _dot(g, s1_scr[...])
            + p1_scr[pl.ds(r, _BM), :].astype(jnp.float32), 0.0).astype(_BF)
        s2_scr[pl.ds(r, _BM), :] = (_dot(h, w2_ref[...].astype(_BF))
                                    + b2_ref[...]).astype(_BF)
        p2_scr[pl.ds(r, _BM), :] = _dot(h, sw2_ref[...].astype(_BF)).astype(_BF)

    @pl.when(i > _NB)
    def _pass_2():
        r = (i - 1 - _NB) * _BM
        g = g_ref[...].astype(_BF)
        o_ref[...] = (_dot(g, s2_scr[...])
                      + p2_scr[pl.ds(r, _BM), :].astype(jnp.float32))


@jax.jit
def kernel(input, G, W1, SW1, b1, W2, SW2, b2):
    x = input
    b1r = b1.reshape(1, _D)
    b2r = b2.reshape(1, _D)

    inv = lambda i: (0, 0)
    return pl.pallas_call(
        _body,
        grid=(2 * _NB + 1,),
        in_specs=[
            pl.BlockSpec((_N, _D), inv),
            pl.BlockSpec((_BM, _N),
                         lambda i: (jnp.maximum(i - 1, 0) % _NB, 0)),
            pl.BlockSpec((_D, _D), inv),
            pl.BlockSpec((_D, _D), inv),
            pl.BlockSpec((1, _D), inv),
            pl.BlockSpec((_D, _D), inv),
            pl.BlockSpec((_D, _D), inv),
            pl.BlockSpec((1, _D), inv),
        ],
        out_specs=pl.BlockSpec((_BM, _D),
                               lambda i: (jnp.maximum(i - 1 - _NB, 0), 0)),
        out_shape=jax.ShapeDtypeStruct((_N, _D), jnp.float32),
        scratch_shapes=[
            pltpu.VMEM((_N, _D), _BF),
            pltpu.VMEM((_N, _D), _BF),
            pltpu.VMEM((_N, _D), _BF),
            pltpu.VMEM((_N, _D), _BF),
        ],
        compiler_params=pltpu.CompilerParams(
            dimension_semantics=("arbitrary",)),
    )(x, G, W1, SW1, b1r, W2, SW2, b2r)
